# baseline (speedup 1.0000x reference)
import functools
import jax
import jax.numpy as jnp
from jax import lax
from jax.experimental import pallas as pl
from jax.experimental.pallas import tpu as pltpu
from jax.experimental.pallas import tpu_sc as plsc

EPS = 1e-5

N = 10000
D = 128
H = 64
E = 320000
NC = 2
NS = 16
NW = NC * NS
CH = 128
GRP = 3
NCHUNK = E // CH
BASE = NCHUNK // NW
EXTRA = NCHUNK % NW
RNDS = BASE // GRP
KMAX = BASE + 1
AW = 8
NP = N + 112
RPT = NP // NS


def _bn(v, g, be):
    m = jnp.mean(v, axis=0, keepdims=True)
    var = jnp.mean((v - m) * (v - m), axis=0, keepdims=True)
    return (v - m) / jnp.sqrt(var + EPS) * g + be


def _tc_a1_body(x_ref, wl_ref, y_ref):
    y = lax.dot_general(x_ref[...], wl_ref[...], (((1,), (1,)), ((), ())),
                        preferred_element_type=jnp.float32)
    y_ref[0:N, :] = y
    y_ref[N:NP, :] = jnp.zeros((NP - N, H), jnp.float32)


def _tc_a1(x, wl):
    return pl.pallas_call(
        _tc_a1_body,
        out_shape=jax.ShapeDtypeStruct((NP, H), jnp.float32),
    )(x, wl)


def _tc_a2_body(x_ref, wr_ref, b_ref, r_ref):
    r_ref[...] = lax.dot_general(x_ref[...], wr_ref[...],
                                 (((1,), (1,)), ((), ())),
                                 preferred_element_type=jnp.float32) + b_ref[...]


def _tc_a2(x, wr, b):
    return pl.pallas_call(
        _tc_a2_body,
        out_shape=jax.ShapeDtypeStruct((N, H), jnp.float32),
    )(x, wr, b)


def _tc_b_body(acc_ref, deg16_ref, r1_ref, wl_ref, g_ref,
               be_ref, y2_ref, h1_ref, inv16_ref):
    deg = (deg16_ref[0, 0:N, :] + deg16_ref[1, 0:N, :])[:, 0:1]
    invdeg = 1.0 / jnp.maximum(deg, 1.0)
    pre = (acc_ref[0, 0:N, :] + acc_ref[1, 0:N, :]) * invdeg + r1_ref[...]
    h1 = jnp.maximum(_bn(pre, g_ref[...], be_ref[...]), 0.0)
    y2_ref[0:N, :] = lax.dot_general(h1, wl_ref[...], (((1,), (1,)), ((), ())),
                                     preferred_element_type=jnp.float32)
    y2_ref[N:NP, :] = jnp.zeros((NP - N, H), jnp.float32)
    h1_ref[...] = h1
    inv16_ref[0:N, :] = jnp.broadcast_to(invdeg, (N, AW))
    inv16_ref[N:NP, :] = jnp.zeros((NP - N, AW), jnp.float32)


def _tc_b(acc1, deg16, r1, wl, g, be):
    return pl.pallas_call(
        _tc_b_body,
        out_shape=[jax.ShapeDtypeStruct((NP, H), jnp.float32),
                   jax.ShapeDtypeStruct((N, H), jnp.float32),
                   jax.ShapeDtypeStruct((NP, AW), jnp.float32)],
    )(acc1, deg16, r1, wl, g, be)


def _tc_b2_body(h1_ref, wr_ref, b_ref, r2_ref):
    r2_ref[...] = lax.dot_general(h1_ref[...], wr_ref[...],
                                  (((1,), (1,)), ((), ())),
                                  preferred_element_type=jnp.float32) + b_ref[...]


def _tc_b2(h1, wr, b):
    return pl.pallas_call(
        _tc_b2_body,
        out_shape=jax.ShapeDtypeStruct((N, H), jnp.float32),
    )(h1, wr, b)


def _tc_c_body(acc_ref, w16_ref, inv16_ref, r2_ref, wl3_ref, wr3_ref, b3_ref,
               g_ref, be_ref, wc1_ref, bc1_ref, wc2_ref, bc2_ref, out_ref):
    invdeg = inv16_ref[0:N, :][:, 0:1]
    pre = (acc_ref[0, 0:N, :] + acc_ref[1, 0:N, :]) * invdeg + r2_ref[...]
    h2 = jnp.maximum(_bn(pre, g_ref[...], be_ref[...]), 0.0)
    y3 = lax.dot_general(h2, wl3_ref[...], (((1,), (1,)), ((), ())),
                         preferred_element_type=jnp.float32)
    z3 = lax.dot_general(h2, wr3_ref[...], (((1,), (1,)), ((), ())),
                         preferred_element_type=jnp.float32)
    w = (w16_ref[0, 0:N, :] + w16_ref[1, 0:N, :])[:, 0:1]
    s = (jnp.sum(y3 * w) + jnp.sum(z3)) * (1.0 / N) + b3_ref[0, 0]
    hh = jnp.maximum(s * wc1_ref[...] + bc1_ref[...], 0.0)
    logit = jnp.sum(hh * wc2_ref[...]) + bc2_ref[0, 0]
    out_ref[...] = (1.0 / (1.0 + jnp.exp(-logit))).reshape(1, 1)


def _tc_c(acc2, w16, inv16, r2, wl3, wr3, b3, g, be, wc1, bc1, wc2, bc2):
    return pl.pallas_call(
        _tc_c_body,
        out_shape=jax.ShapeDtypeStruct((1, 1), jnp.float32),
    )(acc2, w16, inv16, r2, wl3, wr3, b3, g, be, wc1, bc1, wc2, bc2)


_MESH = plsc.VectorSubcoreMesh(core_axis_name="c", subcore_axis_name="s",
                               num_cores=NC, num_subcores=NS)


def _sc1_body(y_hbm, src_hbm, dst_hbm, zrow_hbm, zaw_hbm, ones_hbm,
              acc_out, deg_out,
              idx_s, idx_d, rows, ones_v, acc_sh, deg_sh,
              sem_g0, sem_g1, sem_s0, sem_s1, sem_d):
    c = lax.axis_index("c")
    s = lax.axis_index("s")
    wid = s * NC + c
    row0 = s * RPT
    start = wid * BASE + jnp.minimum(wid, EXTRA)
    has_extra = wid < EXTRA
    pltpu.sync_copy(zrow_hbm, acc_sh.at[pl.ds(row0, RPT)])
    pltpu.sync_copy(zaw_hbm, deg_sh.at[pl.ds(row0, RPT)])
    pltpu.sync_copy(ones_hbm, ones_v)
    pltpu.sync_copy(src_hbm.at[pl.ds(start, BASE)], idx_s.at[pl.ds(0, BASE)])
    pltpu.sync_copy(dst_hbm.at[pl.ds(start, BASE)], idx_d.at[pl.ds(0, BASE)])

    @pl.when(has_extra)
    def _():
        pltpu.sync_copy(src_hbm.at[pl.ds(start + BASE, 1)],
                        idx_s.at[pl.ds(BASE, 1)])
        pltpu.sync_copy(dst_hbm.at[pl.ds(start + BASE, 1)],
                        idx_d.at[pl.ds(BASE, 1)])

    plsc.subcore_barrier()

    sems_g = (sem_g0, sem_g1)
    sems_s = (sem_s0, sem_s1)

    for b in range(GRP):
        pltpu.async_copy(y_hbm.at[idx_s.at[b]], rows.at[b], sem_g0)

    def round_(r, p):
        base = p * GRP
        ob = (1 - p) * GRP
        @pl.when(r + 1 < RNDS)
        def _():
            @pl.when(r >= 1)
            def _():
                for b in range(GRP):
                    pltpu.make_async_copy(
                        rows.at[ob + b], acc_sh.at[idx_d.at[0]],
                        sems_s[1 - p]).wait()
            for b in range(GRP):
                pltpu.async_copy(y_hbm.at[idx_s.at[(r + 1) * GRP + b]],
                                 rows.at[ob + b], sems_g[1 - p])
        for b in range(GRP):
            pltpu.make_async_copy(
                y_hbm.at[idx_s.at[r * GRP + b]], rows.at[base + b],
                sems_g[p]).wait()
        for b in range(GRP):
            j = r * GRP + b
            pltpu.async_copy(rows.at[base + b], acc_sh.at[idx_d.at[j]],
                             sems_s[p], add=True)
            pltpu.async_copy(ones_v, deg_sh.at[idx_d.at[j]], sem_d, add=True)

    def body(r, carry):
        @pl.when(r % 2 == 0)
        def _():
            round_(r, 0)

        @pl.when(r % 2 == 1)
        def _():
            round_(r, 1)
        return carry

    lax.fori_loop(0, RNDS, body, 0)
    for b in range(GRP):
        pltpu.make_async_copy(rows.at[b], acc_sh.at[idx_d.at[0]], sem_s0).wait()
        pltpu.make_async_copy(rows.at[GRP + b], acc_sh.at[idx_d.at[0]],
                              sem_s1).wait()

    @pl.when(has_extra)
    def _():
        pltpu.async_copy(y_hbm.at[idx_s.at[BASE]], rows.at[0], sem_g0)
        pltpu.make_async_copy(y_hbm.at[idx_s.at[BASE]], rows.at[0],
                              sem_g0).wait()
        pltpu.sync_copy(rows.at[0], acc_sh.at[idx_d.at[BASE]], add=True)
        pltpu.sync_copy(ones_v, deg_sh.at[idx_d.at[BASE]], add=True)

    def dbody(j, carry):
        pltpu.make_async_copy(ones_v, deg_sh.at[idx_d.at[0]], sem_d).wait()
        return carry

    lax.fori_loop(0, BASE, dbody, 0)
    plsc.subcore_barrier()
    pltpu.sync_copy(acc_sh.at[pl.ds(row0, RPT)], acc_out.at[c, pl.ds(row0, RPT)])
    pltpu.sync_copy(deg_sh.at[pl.ds(row0, RPT)], deg_out.at[c, pl.ds(row0, RPT)])


def _sc2_body(y_hbm, inv_hbm, src_hbm, dst_hbm, zrow_hbm, zaw_hbm,
              acc_out, w_out,
              idx_s, idx_d, rows, aux, acc_sh, w_sh,
              sem_g0, sem_g1, sem_s0, sem_s1, sem_t0, sem_t1):
    c = lax.axis_index("c")
    s = lax.axis_index("s")
    wid = s * NC + c
    row0 = s * RPT
    start = wid * BASE + jnp.minimum(wid, EXTRA)
    has_extra = wid < EXTRA
    pltpu.sync_copy(zrow_hbm, acc_sh.at[pl.ds(row0, RPT)])
    pltpu.sync_copy(zaw_hbm, w_sh.at[pl.ds(row0, RPT)])
    pltpu.sync_copy(src_hbm.at[pl.ds(start, BASE)], idx_s.at[pl.ds(0, BASE)])
    pltpu.sync_copy(dst_hbm.at[pl.ds(start, BASE)], idx_d.at[pl.ds(0, BASE)])

    @pl.when(has_extra)
    def _():
        pltpu.sync_copy(src_hbm.at[pl.ds(start + BASE, 1)],
                        idx_s.at[pl.ds(BASE, 1)])
        pltpu.sync_copy(dst_hbm.at[pl.ds(start + BASE, 1)],
                        idx_d.at[pl.ds(BASE, 1)])

    plsc.subcore_barrier()

    sems_g = (sem_g0, sem_g1)
    sems_s = (sem_s0, sem_s1)
    sems_t = (sem_t0, sem_t1)

    for b in range(GRP):
        pltpu.async_copy(y_hbm.at[idx_s.at[b]], rows.at[b], sem_g0)
        pltpu.async_copy(inv_hbm.at[idx_d.at[b]], aux.at[b], sem_g0)

    def round_(r, p):
        base = p * GRP
        ob = (1 - p) * GRP
        @pl.when(r + 1 < RNDS)
        def _():
            @pl.when(r >= 1)
            def _():
                for b in range(GRP):
                    pltpu.make_async_copy(
                        rows.at[ob + b], acc_sh.at[idx_d.at[0]],
                        sems_s[1 - p]).wait()
                    pltpu.make_async_copy(
                        aux.at[ob + b], w_sh.at[idx_s.at[0]],
                        sems_t[1 - p]).wait()
            for b in range(GRP):
                j = (r + 1) * GRP + b
                pltpu.async_copy(y_hbm.at[idx_s.at[j]], rows.at[ob + b],
                                 sems_g[1 - p])
                pltpu.async_copy(inv_hbm.at[idx_d.at[j]], aux.at[ob + b],
                                 sems_g[1 - p])
        for b in range(GRP):
            j = r * GRP + b
            pltpu.make_async_copy(
                y_hbm.at[idx_s.at[j]], rows.at[base + b], sems_g[p]).wait()
            pltpu.make_async_copy(
                inv_hbm.at[idx_d.at[j]], aux.at[base + b], sems_g[p]).wait()
        for b in range(GRP):
            j = r * GRP + b
            pltpu.async_copy(rows.at[base + b], acc_sh.at[idx_d.at[j]],
                             sems_s[p], add=True)
            pltpu.async_copy(aux.at[base + b], w_sh.at[idx_s.at[j]],
                             sems_t[p], add=True)

    def body(r, carry):
        @pl.when(r % 2 == 0)
        def _():
            round_(r, 0)

        @pl.when(r % 2 == 1)
        def _():
            round_(r, 1)
        return carry

    lax.fori_loop(0, RNDS, body, 0)
    for b in range(GRP):
        pltpu.make_async_copy(rows.at[b], acc_sh.at[idx_d.at[0]], sem_s0).wait()
        pltpu.make_async_copy(rows.at[GRP + b], acc_sh.at[idx_d.at[0]],
                              sem_s1).wait()
        pltpu.make_async_copy(aux.at[b], w_sh.at[idx_s.at[0]], sem_t0).wait()
        pltpu.make_async_copy(aux.at[GRP + b], w_sh.at[idx_s.at[0]],
                              sem_t1).wait()

    @pl.when(has_extra)
    def _():
        pltpu.async_copy(y_hbm.at[idx_s.at[BASE]], rows.at[0], sem_g0)
        pltpu.make_async_copy(y_hbm.at[idx_s.at[BASE]], rows.at[0],
                              sem_g0).wait()
        pltpu.sync_copy(rows.at[0], acc_sh.at[idx_d.at[BASE]], add=True)
        pltpu.async_copy(inv_hbm.at[idx_d.at[BASE]], aux.at[0], sem_g0)
        pltpu.make_async_copy(inv_hbm.at[idx_d.at[BASE]], aux.at[0],
                              sem_g0).wait()
        pltpu.sync_copy(aux.at[0], w_sh.at[idx_s.at[BASE]], add=True)

    plsc.subcore_barrier()
    pltpu.sync_copy(acc_sh.at[pl.ds(row0, RPT)], acc_out.at[c, pl.ds(row0, RPT)])
    pltpu.sync_copy(w_sh.at[pl.ds(row0, RPT)], w_out.at[c, pl.ds(row0, RPT)])


@functools.partial(
    pl.kernel,
    out_type=[jax.ShapeDtypeStruct((NC, NP, H), jnp.float32),
              jax.ShapeDtypeStruct((NC, NP, AW), jnp.float32)],
    mesh=_MESH,
    scratch_types=[
        pltpu.VMEM((KMAX, CH), jnp.int32),
        pltpu.VMEM((KMAX, CH), jnp.int32),
        pltpu.VMEM((2 * GRP, CH, H), jnp.float32),
        pltpu.VMEM((CH, AW), jnp.float32),
        pltpu.VMEM_SHARED((NP, H), jnp.float32),
        pltpu.VMEM_SHARED((NP, AW), jnp.float32),
        pltpu.SemaphoreType.DMA,
        pltpu.SemaphoreType.DMA,
        pltpu.SemaphoreType.DMA,
        pltpu.SemaphoreType.DMA,
        pltpu.SemaphoreType.DMA,
    ],
    compiler_params=pltpu.CompilerParams(use_tc_tiling_on_sc=False),
)
def _sc1(*refs):
    _sc1_body(*refs)


@functools.partial(
    pl.kernel,
    out_type=[jax.ShapeDtypeStruct((NC, NP, H), jnp.float32),
              jax.ShapeDtypeStruct((NC, NP, AW), jnp.float32)],
    mesh=_MESH,
    scratch_types=[
        pltpu.VMEM((KMAX, CH), jnp.int32),
        pltpu.VMEM((KMAX, CH), jnp.int32),
        pltpu.VMEM((2 * GRP, CH, H), jnp.float32),
        pltpu.VMEM((2 * GRP, CH, AW), jnp.float32),
        pltpu.VMEM_SHARED((NP, H), jnp.float32),
        pltpu.VMEM_SHARED((NP, AW), jnp.float32),
        pltpu.SemaphoreType.DMA,
        pltpu.SemaphoreType.DMA,
        pltpu.SemaphoreType.DMA,
        pltpu.SemaphoreType.DMA,
        pltpu.SemaphoreType.DMA,
        pltpu.SemaphoreType.DMA,
    ],
    compiler_params=pltpu.CompilerParams(use_tc_tiling_on_sc=False),
)
def _sc2(*refs):
    _sc2_body(*refs)


def kernel(x, edge_index, Wl1, Wr1, b1, Wl2, Wr2, b2, Wl3, Wr3, b3,
           g1, be1, g2, be2, Wc1, bc1, Wc2, bc2):
    ei = edge_index.astype(jnp.int32)
    src2 = ei[0].reshape(NCHUNK, CH)
    dst2 = ei[1].reshape(NCHUNK, CH)

    zrow = jnp.zeros((RPT, H), jnp.float32)
    zaw = jnp.zeros((RPT, AW), jnp.float32)
    onesaw = jnp.ones((CH, AW), jnp.float32)

    b1r = b1.reshape(1, H)
    b2r = b2.reshape(1, H)
    b3r = b3.reshape(1, 1)
    g1r, be1r = g1.reshape(1, H), be1.reshape(1, H)
    g2r, be2r = g2.reshape(1, H), be2.reshape(1, H)
    wc1r = Wc1.reshape(1, H)
    bc1r = bc1.reshape(1, H)
    wc2r = Wc2.reshape(1, H)
    bc2r = bc2.reshape(1, 1)

    y1 = _tc_a1(x, Wl1)
    acc1, deg16 = _sc1(y1, src2, dst2, zrow, zaw, onesaw)
    r1 = _tc_a2(x, Wr1, b1r)
    y2, h1, inv16 = _tc_b(acc1, deg16, r1, Wl2, g1r, be1r)
    acc2, w16 = _sc2(y2, inv16, src2, dst2, zrow, zaw)
    r2 = _tc_b2(h1, Wr2, b2r)
    out = _tc_c(acc2, w16, inv16, r2, Wl3, Wr3, b3r, g2r, be2r,
                wc1r, bc1r, wc2r, bc2r)
    return out


# speedup vs baseline: 1.1624x; 1.1624x over previous
import functools
import jax
import jax.numpy as jnp
from jax import lax
from jax.experimental import pallas as pl
from jax.experimental.pallas import tpu as pltpu
from jax.experimental.pallas import tpu_sc as plsc

EPS = 1e-5

N = 10000
D = 128
H = 64
E = 320000
NC = 2
NS = 16
NW = NC * NS
CH = 128
GRP = 3
NCHUNK = E // CH
BASE = NCHUNK // NW
EXTRA = NCHUNK % NW
RNDS = BASE // GRP
KMAX = BASE + 1
AW = 8
NP = N + 112
RPT = NP // NS


def _bn(v, g, be):
    m = jnp.mean(v, axis=0, keepdims=True)
    var = jnp.mean((v - m) * (v - m), axis=0, keepdims=True)
    return (v - m) / jnp.sqrt(var + EPS) * g + be


def _tc_a1_body(x_ref, wl_ref, y_ref):
    y = lax.dot_general(x_ref[...], wl_ref[...], (((1,), (1,)), ((), ())),
                        preferred_element_type=jnp.float32)
    y_ref[0:N, :] = y.astype(jnp.bfloat16)
    y_ref[N:NP, :] = jnp.zeros((NP - N, H), jnp.bfloat16)


def _tc_a1(x, wl):
    return pl.pallas_call(
        _tc_a1_body,
        out_shape=jax.ShapeDtypeStruct((NP, H), jnp.bfloat16),
    )(x, wl)


def _tc_a2_body(x_ref, wr_ref, b_ref, r_ref):
    r_ref[...] = lax.dot_general(x_ref[...], wr_ref[...],
                                 (((1,), (1,)), ((), ())),
                                 preferred_element_type=jnp.float32) + b_ref[...]


def _tc_a2(x, wr, b):
    return pl.pallas_call(
        _tc_a2_body,
        out_shape=jax.ShapeDtypeStruct((N, H), jnp.float32),
    )(x, wr, b)


def _tc_b_body(acc_ref, deg16_ref, r1_ref, wl_ref, g_ref,
               be_ref, y2_ref, h1_ref, inv16_ref):
    deg = (deg16_ref[0, 0:N, :] + deg16_ref[1, 0:N, :])[:, 0:1]
    invdeg = 1.0 / jnp.maximum(deg, 1.0)
    agg = (acc_ref[0, 0:N, :].astype(jnp.float32)
           + acc_ref[1, 0:N, :].astype(jnp.float32))
    pre = agg * invdeg + r1_ref[...]
    h1 = jnp.maximum(_bn(pre, g_ref[...], be_ref[...]), 0.0)
    y2 = lax.dot_general(h1, wl_ref[...], (((1,), (1,)), ((), ())),
                         preferred_element_type=jnp.float32)
    y2_ref[0:N, :] = y2.astype(jnp.bfloat16)
    y2_ref[N:NP, :] = jnp.zeros((NP - N, H), jnp.bfloat16)
    h1_ref[...] = h1
    inv16_ref[0:N, :] = jnp.broadcast_to(invdeg, (N, AW))
    inv16_ref[N:NP, :] = jnp.zeros((NP - N, AW), jnp.float32)


def _tc_b(acc1, deg16, r1, wl, g, be):
    return pl.pallas_call(
        _tc_b_body,
        out_shape=[jax.ShapeDtypeStruct((NP, H), jnp.bfloat16),
                   jax.ShapeDtypeStruct((N, H), jnp.float32),
                   jax.ShapeDtypeStruct((NP, AW), jnp.float32)],
    )(acc1, deg16, r1, wl, g, be)


def _tc_b2_body(h1_ref, wr_ref, b_ref, r2_ref):
    r2_ref[...] = lax.dot_general(h1_ref[...], wr_ref[...],
                                  (((1,), (1,)), ((), ())),
                                  preferred_element_type=jnp.float32) + b_ref[...]


def _tc_b2(h1, wr, b):
    return pl.pallas_call(
        _tc_b2_body,
        out_shape=jax.ShapeDtypeStruct((N, H), jnp.float32),
    )(h1, wr, b)


def _tc_c_body(acc_ref, w16_ref, inv16_ref, r2_ref, wl3_ref, wr3_ref, b3_ref,
               g_ref, be_ref, wc1_ref, bc1_ref, wc2_ref, bc2_ref, out_ref):
    invdeg = inv16_ref[0:N, :][:, 0:1]
    agg = (acc_ref[0, 0:N, :].astype(jnp.float32)
           + acc_ref[1, 0:N, :].astype(jnp.float32))
    pre = agg * invdeg + r2_ref[...]
    h2 = jnp.maximum(_bn(pre, g_ref[...], be_ref[...]), 0.0)
    y3 = lax.dot_general(h2, wl3_ref[...], (((1,), (1,)), ((), ())),
                         preferred_element_type=jnp.float32)
    z3 = lax.dot_general(h2, wr3_ref[...], (((1,), (1,)), ((), ())),
                         preferred_element_type=jnp.float32)
    w = (w16_ref[0, 0:N, :] + w16_ref[1, 0:N, :])[:, 0:1]
    s = (jnp.sum(y3 * w) + jnp.sum(z3)) * (1.0 / N) + b3_ref[0, 0]
    hh = jnp.maximum(s * wc1_ref[...] + bc1_ref[...], 0.0)
    logit = jnp.sum(hh * wc2_ref[...]) + bc2_ref[0, 0]
    out_ref[...] = (1.0 / (1.0 + jnp.exp(-logit))).reshape(1, 1)


def _tc_c(acc2, w16, inv16, r2, wl3, wr3, b3, g, be, wc1, bc1, wc2, bc2):
    return pl.pallas_call(
        _tc_c_body,
        out_shape=jax.ShapeDtypeStruct((1, 1), jnp.float32),
    )(acc2, w16, inv16, r2, wl3, wr3, b3, g, be, wc1, bc1, wc2, bc2)


_MESH = plsc.VectorSubcoreMesh(core_axis_name="c", subcore_axis_name="s",
                               num_cores=NC, num_subcores=NS)


def _sc1_body(y_hbm, src_hbm, dst_hbm, zrow_hbm, zaw_hbm, ones_hbm,
              acc_out, deg_out,
              idx_s, idx_d, rows, ones_v, acc_sh, deg_sh,
              sem_g0, sem_g1, sem_s0, sem_s1, sem_d):
    c = lax.axis_index("c")
    s = lax.axis_index("s")
    wid = s * NC + c
    row0 = s * RPT
    start = wid * BASE + jnp.minimum(wid, EXTRA)
    has_extra = wid < EXTRA
    pltpu.sync_copy(zrow_hbm, acc_sh.at[pl.ds(row0, RPT)])
    pltpu.sync_copy(zaw_hbm, deg_sh.at[pl.ds(row0, RPT)])
    pltpu.sync_copy(ones_hbm, ones_v)
    pltpu.sync_copy(src_hbm.at[pl.ds(start, BASE)], idx_s.at[pl.ds(0, BASE)])
    pltpu.sync_copy(dst_hbm.at[pl.ds(start, BASE)], idx_d.at[pl.ds(0, BASE)])

    @pl.when(has_extra)
    def _():
        pltpu.sync_copy(src_hbm.at[pl.ds(start + BASE, 1)],
                        idx_s.at[pl.ds(BASE, 1)])
        pltpu.sync_copy(dst_hbm.at[pl.ds(start + BASE, 1)],
                        idx_d.at[pl.ds(BASE, 1)])

    plsc.subcore_barrier()

    sems_g = (sem_g0, sem_g1)
    sems_s = (sem_s0, sem_s1)

    for b in range(GRP):
        pltpu.async_copy(y_hbm.at[idx_s.at[b]], rows.at[b], sem_g0)

    def round_(r, p):
        base = p * GRP
        ob = (1 - p) * GRP
        @pl.when(r + 1 < RNDS)
        def _():
            @pl.when(r >= 1)
            def _():
                for b in range(GRP):
                    pltpu.make_async_copy(
                        rows.at[ob + b], acc_sh.at[idx_d.at[0]],
                        sems_s[1 - p]).wait()
            for b in range(GRP):
                pltpu.async_copy(y_hbm.at[idx_s.at[(r + 1) * GRP + b]],
                                 rows.at[ob + b], sems_g[1 - p])
        for b in range(GRP):
            pltpu.make_async_copy(
                y_hbm.at[idx_s.at[r * GRP + b]], rows.at[base + b],
                sems_g[p]).wait()
        for b in range(GRP):
            j = r * GRP + b
            pltpu.async_copy(rows.at[base + b], acc_sh.at[idx_d.at[j]],
                             sems_s[p], add=True)
            pltpu.async_copy(ones_v, deg_sh.at[idx_d.at[j]], sem_d, add=True)

    def body(r, carry):
        @pl.when(r % 2 == 0)
        def _():
            round_(r, 0)

        @pl.when(r % 2 == 1)
        def _():
            round_(r, 1)
        return carry

    lax.fori_loop(0, RNDS, body, 0)
    for b in range(GRP):
        pltpu.make_async_copy(rows.at[b], acc_sh.at[idx_d.at[0]], sem_s0).wait()
        pltpu.make_async_copy(rows.at[GRP + b], acc_sh.at[idx_d.at[0]],
                              sem_s1).wait()

    @pl.when(has_extra)
    def _():
        pltpu.async_copy(y_hbm.at[idx_s.at[BASE]], rows.at[0], sem_g0)
        pltpu.make_async_copy(y_hbm.at[idx_s.at[BASE]], rows.at[0],
                              sem_g0).wait()
        pltpu.sync_copy(rows.at[0], acc_sh.at[idx_d.at[BASE]], add=True)
        pltpu.sync_copy(ones_v, deg_sh.at[idx_d.at[BASE]], add=True)

    def dbody(j, carry):
        pltpu.make_async_copy(ones_v, deg_sh.at[idx_d.at[0]], sem_d).wait()
        return carry

    lax.fori_loop(0, BASE, dbody, 0)
    plsc.subcore_barrier()
    pltpu.sync_copy(acc_sh.at[pl.ds(row0, RPT)], acc_out.at[c, pl.ds(row0, RPT)])
    pltpu.sync_copy(deg_sh.at[pl.ds(row0, RPT)], deg_out.at[c, pl.ds(row0, RPT)])


def _sc2_body(y_hbm, inv_hbm, src_hbm, dst_hbm, zrow_hbm, zaw_hbm,
              acc_out, w_out,
              idx_s, idx_d, rows, aux, acc_sh, w_sh,
              sem_g0, sem_g1, sem_s0, sem_s1, sem_t0, sem_t1):
    c = lax.axis_index("c")
    s = lax.axis_index("s")
    wid = s * NC + c
    row0 = s * RPT
    start = wid * BASE + jnp.minimum(wid, EXTRA)
    has_extra = wid < EXTRA
    pltpu.sync_copy(zrow_hbm, acc_sh.at[pl.ds(row0, RPT)])
    pltpu.sync_copy(zaw_hbm, w_sh.at[pl.ds(row0, RPT)])
    pltpu.sync_copy(src_hbm.at[pl.ds(start, BASE)], idx_s.at[pl.ds(0, BASE)])
    pltpu.sync_copy(dst_hbm.at[pl.ds(start, BASE)], idx_d.at[pl.ds(0, BASE)])

    @pl.when(has_extra)
    def _():
        pltpu.sync_copy(src_hbm.at[pl.ds(start + BASE, 1)],
                        idx_s.at[pl.ds(BASE, 1)])
        pltpu.sync_copy(dst_hbm.at[pl.ds(start + BASE, 1)],
                        idx_d.at[pl.ds(BASE, 1)])

    plsc.subcore_barrier()

    sems_g = (sem_g0, sem_g1)
    sems_s = (sem_s0, sem_s1)
    sems_t = (sem_t0, sem_t1)

    for b in range(GRP):
        pltpu.async_copy(y_hbm.at[idx_s.at[b]], rows.at[b], sem_g0)
        pltpu.async_copy(inv_hbm.at[idx_d.at[b]], aux.at[b], sem_g0)

    def round_(r, p):
        base = p * GRP
        ob = (1 - p) * GRP
        @pl.when(r + 1 < RNDS)
        def _():
            @pl.when(r >= 1)
            def _():
                for b in range(GRP):
                    pltpu.make_async_copy(
                        rows.at[ob + b], acc_sh.at[idx_d.at[0]],
                        sems_s[1 - p]).wait()
                    pltpu.make_async_copy(
                        aux.at[ob + b], w_sh.at[idx_s.at[0]],
                        sems_t[1 - p]).wait()
            for b in range(GRP):
                j = (r + 1) * GRP + b
                pltpu.async_copy(y_hbm.at[idx_s.at[j]], rows.at[ob + b],
                                 sems_g[1 - p])
                pltpu.async_copy(inv_hbm.at[idx_d.at[j]], aux.at[ob + b],
                                 sems_g[1 - p])
        for b in range(GRP):
            j = r * GRP + b
            pltpu.make_async_copy(
                y_hbm.at[idx_s.at[j]], rows.at[base + b], sems_g[p]).wait()
            pltpu.make_async_copy(
                inv_hbm.at[idx_d.at[j]], aux.at[base + b], sems_g[p]).wait()
        for b in range(GRP):
            j = r * GRP + b
            pltpu.async_copy(rows.at[base + b], acc_sh.at[idx_d.at[j]],
                             sems_s[p], add=True)
            pltpu.async_copy(aux.at[base + b], w_sh.at[idx_s.at[j]],
                             sems_t[p], add=True)

    def body(r, carry):
        @pl.when(r % 2 == 0)
        def _():
            round_(r, 0)

        @pl.when(r % 2 == 1)
        def _():
            round_(r, 1)
        return carry

    lax.fori_loop(0, RNDS, body, 0)
    for b in range(GRP):
        pltpu.make_async_copy(rows.at[b], acc_sh.at[idx_d.at[0]], sem_s0).wait()
        pltpu.make_async_copy(rows.at[GRP + b], acc_sh.at[idx_d.at[0]],
                              sem_s1).wait()
        pltpu.make_async_copy(aux.at[b], w_sh.at[idx_s.at[0]], sem_t0).wait()
        pltpu.make_async_copy(aux.at[GRP + b], w_sh.at[idx_s.at[0]],
                              sem_t1).wait()

    @pl.when(has_extra)
    def _():
        pltpu.async_copy(y_hbm.at[idx_s.at[BASE]], rows.at[0], sem_g0)
        pltpu.make_async_copy(y_hbm.at[idx_s.at[BASE]], rows.at[0],
                              sem_g0).wait()
        pltpu.sync_copy(rows.at[0], acc_sh.at[idx_d.at[BASE]], add=True)
        pltpu.async_copy(inv_hbm.at[idx_d.at[BASE]], aux.at[0], sem_g0)
        pltpu.make_async_copy(inv_hbm.at[idx_d.at[BASE]], aux.at[0],
                              sem_g0).wait()
        pltpu.sync_copy(aux.at[0], w_sh.at[idx_s.at[BASE]], add=True)

    plsc.subcore_barrier()
    pltpu.sync_copy(acc_sh.at[pl.ds(row0, RPT)], acc_out.at[c, pl.ds(row0, RPT)])
    pltpu.sync_copy(w_sh.at[pl.ds(row0, RPT)], w_out.at[c, pl.ds(row0, RPT)])


@functools.partial(
    pl.kernel,
    out_type=[jax.ShapeDtypeStruct((NC, NP, H), jnp.bfloat16),
              jax.ShapeDtypeStruct((NC, NP, AW), jnp.float32)],
    mesh=_MESH,
    scratch_types=[
        pltpu.VMEM((KMAX, CH), jnp.int32),
        pltpu.VMEM((KMAX, CH), jnp.int32),
        pltpu.VMEM((2 * GRP, CH, H), jnp.bfloat16),
        pltpu.VMEM((CH, AW), jnp.float32),
        pltpu.VMEM_SHARED((NP, H), jnp.bfloat16),
        pltpu.VMEM_SHARED((NP, AW), jnp.float32),
        pltpu.SemaphoreType.DMA,
        pltpu.SemaphoreType.DMA,
        pltpu.SemaphoreType.DMA,
        pltpu.SemaphoreType.DMA,
        pltpu.SemaphoreType.DMA,
    ],
    compiler_params=pltpu.CompilerParams(use_tc_tiling_on_sc=False),
)
def _sc1(*refs):
    _sc1_body(*refs)


@functools.partial(
    pl.kernel,
    out_type=[jax.ShapeDtypeStruct((NC, NP, H), jnp.bfloat16),
              jax.ShapeDtypeStruct((NC, NP, AW), jnp.float32)],
    mesh=_MESH,
    scratch_types=[
        pltpu.VMEM((KMAX, CH), jnp.int32),
        pltpu.VMEM((KMAX, CH), jnp.int32),
        pltpu.VMEM((2 * GRP, CH, H), jnp.bfloat16),
        pltpu.VMEM((2 * GRP, CH, AW), jnp.float32),
        pltpu.VMEM_SHARED((NP, H), jnp.bfloat16),
        pltpu.VMEM_SHARED((NP, AW), jnp.float32),
        pltpu.SemaphoreType.DMA,
        pltpu.SemaphoreType.DMA,
        pltpu.SemaphoreType.DMA,
        pltpu.SemaphoreType.DMA,
        pltpu.SemaphoreType.DMA,
        pltpu.SemaphoreType.DMA,
    ],
    compiler_params=pltpu.CompilerParams(use_tc_tiling_on_sc=False),
)
def _sc2(*refs):
    _sc2_body(*refs)


def kernel(x, edge_index, Wl1, Wr1, b1, Wl2, Wr2, b2, Wl3, Wr3, b3,
           g1, be1, g2, be2, Wc1, bc1, Wc2, bc2):
    ei = edge_index.astype(jnp.int32)
    src2 = ei[0].reshape(NCHUNK, CH)
    dst2 = ei[1].reshape(NCHUNK, CH)

    zrow = jnp.zeros((RPT, H), jnp.bfloat16)
    zaw = jnp.zeros((RPT, AW), jnp.float32)
    onesaw = jnp.ones((CH, AW), jnp.float32)

    b1r = b1.reshape(1, H)
    b2r = b2.reshape(1, H)
    b3r = b3.reshape(1, 1)
    g1r, be1r = g1.reshape(1, H), be1.reshape(1, H)
    g2r, be2r = g2.reshape(1, H), be2.reshape(1, H)
    wc1r = Wc1.reshape(1, H)
    bc1r = bc1.reshape(1, H)
    wc2r = Wc2.reshape(1, H)
    bc2r = bc2.reshape(1, 1)

    y1 = _tc_a1(x, Wl1)
    acc1, deg16 = _sc1(y1, src2, dst2, zrow, zaw, onesaw)
    r1 = _tc_a2(x, Wr1, b1r)
    y2, h1, inv16 = _tc_b(acc1, deg16, r1, Wl2, g1r, be1r)
    acc2, w16 = _sc2(y2, inv16, src2, dst2, zrow, zaw)
    r2 = _tc_b2(h1, Wr2, b2r)

    out = _tc_c(acc2, w16, inv16, r2, Wl3, Wr3, b3r, g2r, be2r,
                wc1r, bc1r, wc2r, bc2r)
    return out
